# Initial kernel scaffold; baseline (speedup 1.0000x reference)
import jax
import jax.numpy as jnp
from jax.experimental import pallas as pl


def kernel(objs, triplets, triplet_type, params):
    raise NotImplementedError("write your pallas kernel here")


# speedup vs baseline: 2.2256x; 2.2256x over previous
import functools

import jax
import jax.numpy as jnp
from jax import lax
from jax.experimental import pallas as pl
from jax.experimental.pallas import tpu as pltpu
from jax.experimental.pallas import tpu_sc as plsc

O_NODES = 10000
T_EDGES = 320000
EMB = 128
H = 128
GDIM = 128
NLAYERS = 5
ATTR_VOCAB = 200

NC = 2
NS = 16
NW = NC * NS
TP = 327680
CH = 128
EW_G = TP // NW
NCH_G = EW_G // CH
EW_S = TP // NS
NCH_S = EW_S // CH

_MM = functools.partial(jnp.dot, precision=lax.Precision.HIGHEST,
                        preferred_element_type=jnp.float32)

@functools.cache
def _mesh():
  return plsc.VectorSubcoreMesh(core_axis_name="c", subcore_axis_name="s",
                                num_cores=NC, num_subcores=NS)


def _wid():
  return lax.axis_index("s") * NC + lax.axis_index("c")


def _gather2_body(ta, tb, ia, ib, out, iva, ivb, ra, rb, sem):
  base = _wid() * EW_G

  def chunk(j, carry):
    off = base + j * CH
    pltpu.sync_copy(ia.at[pl.ds(off, CH)], iva)
    pltpu.sync_copy(ib.at[pl.ds(off, CH)], ivb)
    cpa = pltpu.async_copy(ta.at[iva], ra, sem)
    cpb = pltpu.async_copy(tb.at[ivb], rb, sem)
    cpa.wait()
    cpb.wait()

    def addrow(r, c2):
      for q in range(8):
        sl = pl.ds(q * 16, 16)
        ra[r, sl] = ra[r, sl] + rb[r, sl]
      return c2

    lax.fori_loop(0, CH, addrow, 0)
    pltpu.sync_copy(ra, out.at[pl.ds(off, CH)])
    return carry

  lax.fori_loop(0, NCH_G, chunk, 0)


def _gather3_body(ta, tb, tc_, ia, ib, ic, out, iva, ivb, ivc, ra, rb, rc,
                  sem):
  base = _wid() * EW_G

  def chunk(j, carry):
    off = base + j * CH
    pltpu.sync_copy(ia.at[pl.ds(off, CH)], iva)
    pltpu.sync_copy(ib.at[pl.ds(off, CH)], ivb)
    pltpu.sync_copy(ic.at[pl.ds(off, CH)], ivc)
    cpa = pltpu.async_copy(ta.at[iva], ra, sem)
    cpb = pltpu.async_copy(tb.at[ivb], rb, sem)
    cpc = pltpu.async_copy(tc_.at[ivc], rc, sem)
    cpa.wait()
    cpb.wait()
    cpc.wait()

    def addrow(r, c2):
      for q in range(8):
        sl = pl.ds(q * 16, 16)
        ra[r, sl] = (ra[r, sl] + rb[r, sl]) + rc[r, sl]
      return c2

    lax.fori_loop(0, CH, addrow, 0)
    pltpu.sync_copy(ra, out.at[pl.ds(off, CH)])
    return carry

  lax.fori_loop(0, NCH_G, chunk, 0)


def _sc_gather2(ta, tb, ia, ib):
  k = pl.kernel(
      _gather2_body,
      out_type=jax.ShapeDtypeStruct((TP, H), jnp.float32),
      mesh=_mesh(),
      scratch_types=[
          pltpu.VMEM((CH,), jnp.int32),
          pltpu.VMEM((CH,), jnp.int32),
          pltpu.VMEM((CH, H), jnp.float32),
          pltpu.VMEM((CH, H), jnp.float32),
          pltpu.SemaphoreType.DMA,
      ],
  )
  return k(ta, tb, ia, ib)


def _sc_gather3(ta, tb, tc_, ia, ib, ic):
  k = pl.kernel(
      _gather3_body,
      out_type=jax.ShapeDtypeStruct((TP, H), jnp.float32),
      mesh=_mesh(),
      scratch_types=[
          pltpu.VMEM((CH,), jnp.int32),
          pltpu.VMEM((CH,), jnp.int32),
          pltpu.VMEM((CH,), jnp.int32),
          pltpu.VMEM((CH, H), jnp.float32),
          pltpu.VMEM((CH, H), jnp.float32),
          pltpu.VMEM((CH, H), jnp.float32),
          pltpu.SemaphoreType.DMA,
      ],
  )
  return k(ta, tb, tc_, ia, ib, ic)


def _scatter_body(hm, idx2, out, iv, rv, zb, acc, sem):
  c = lax.axis_index("c")
  sid = lax.axis_index("s")

  def zrow(r, c2):
    for q in range(8):
      zb[r, pl.ds(q * 16, 16)] = jnp.zeros((16,), jnp.float32)
    return c2

  lax.fori_loop(0, 64, zrow, 0)
  row0 = jnp.minimum(sid * 640, O_NODES - 640)

  def zcopy(kk, c2):
    pltpu.sync_copy(zb, acc.at[pl.ds(row0 + kk * 64, 64)])
    return c2

  lax.fori_loop(0, 10, zcopy, 0)
  plsc.subcore_barrier()

  base = sid * EW_S

  def chunk(j, carry):
    off = base + j * CH
    pltpu.sync_copy(idx2.at[c, pl.ds(off, CH)], iv)
    pltpu.sync_copy(hm.at[pl.ds(off, CH)], rv)
    pltpu.sync_copy(rv, acc.at[iv], add=True)
    return carry

  lax.fori_loop(0, NCH_S, chunk, 0)
  plsc.subcore_barrier()
  pltpu.sync_copy(acc.at[pl.ds(row0, 640)], out.at[c].at[pl.ds(row0, 640)])


def _sc_scatter(hm, idx2):
  k = pl.kernel(
      _scatter_body,
      out_type=jax.ShapeDtypeStruct((2, O_NODES, H), jnp.float32),
      mesh=_mesh(),
      scratch_types=[
          pltpu.VMEM((CH,), jnp.int32),
          pltpu.VMEM((CH, H), jnp.float32),
          pltpu.VMEM((64, H), jnp.float32),
          pltpu.VMEM_SHARED((O_NODES, H), jnp.float32),
          pltpu.SemaphoreType.DMA,
      ],
  )
  return k(hm, idx2)


def _counts_body(sarr, oarr, parr, out, sb, ob, pb, cs, co):
  w = _wid()
  base = w * EW_G

  def zero(i, c2):
    sl = pl.ds(i * 16, 16)
    cs[sl] = jnp.zeros((16,), jnp.float32)
    co[sl] = jnp.zeros((16,), jnp.float32)
    return c2

  lax.fori_loop(0, O_NODES // 16, zero, 0)
  pltpu.sync_copy(sarr.at[pl.ds(base, EW_G)], sb)
  pltpu.sync_copy(oarr.at[pl.ds(base, EW_G)], ob)
  pltpu.sync_copy(parr.at[pl.ds(base, EW_G)], pb)

  def step(i, c2):
    sl = pl.ds(i * 16, 16)
    sv = sb[sl]
    ov = ob[sl]
    pv = pb[sl]
    m = jnp.where(pv != 0, 1.0, 0.0).astype(jnp.float32)
    plsc.addupdate_scatter(cs, [sv], m)
    plsc.addupdate_scatter(co, [ov], m)
    return c2

  lax.fori_loop(0, EW_G // 16, step, 0)
  pltpu.sync_copy(cs, out.at[0].at[w])
  pltpu.sync_copy(co, out.at[1].at[w])


def _sc_counts(sarr, oarr, parr):
  k = pl.kernel(
      _counts_body,
      out_type=jax.ShapeDtypeStruct((2, NW, O_NODES), jnp.float32),
      mesh=_mesh(),
      compiler_params=pltpu.CompilerParams(needs_layout_passes=False),
      scratch_types=[
          pltpu.VMEM((EW_G,), jnp.int32),
          pltpu.VMEM((EW_G,), jnp.int32),
          pltpu.VMEM((EW_G,), jnp.int32),
          pltpu.VMEM((O_NODES,), jnp.float32),
          pltpu.VMEM((O_NODES,), jnp.float32),
      ],
  )
  return k(sarr, oarr, parr)


NB = 2000


def _prep_body(o0_ref, o1_ref, tab0_ref, tab1_ref, pe_ref, w1_ref, b1_ref,
               w2ps_ref, w1ps_ref, b2ps_ref, b1s_ref, cin_ref,
               as_ref, ao_ref, p0_ref, ms_ref, cs_ref, cnts_ref, cnto_ref):
  w1 = w1_ref[...]
  tab0 = tab0_ref[...]
  tab1 = tab1_ref[...]
  t0s = _MM(tab0, w1[0:EMB])
  t1s = _MM(tab1, w1[EMB:2 * EMB])
  t0o = _MM(tab0, w1[384:384 + EMB])
  t1o = _MM(tab1, w1[384 + EMB:384 + 2 * EMB])
  iota = lax.broadcasted_iota(jnp.int32, (NB, ATTR_VOCAB), 1)
  oh0 = (o0_ref[...] == iota).astype(jnp.float32)
  oh1 = (o1_ref[...] == iota).astype(jnp.float32)
  as_ref[...] = _MM(oh0, t0s) + _MM(oh1, t1s)
  ao_ref[...] = _MM(oh0, t0o) + _MM(oh1, t1o)
  p0_ref[...] = _MM(pe_ref[...], w1[256:384]) + b1_ref[...]
  for l in range(NLAYERS - 1):
    ms_ref[l] = _MM(w2ps_ref[l], w1ps_ref[l])
    cs_ref[l:l + 1] = _MM(b2ps_ref[l:l + 1], w1ps_ref[l]) + b1s_ref[l:l + 1]
  cin = cin_ref[...]
  cnts_ref[...] = jnp.sum(cin[:, 0:NW], axis=1, keepdims=True)
  cnto_ref[...] = jnp.sum(cin[:, NW:2 * NW], axis=1, keepdims=True)


def _tc_prep(objs0, objs1, tab0, tab1, pred_emb, w1_0, b1_0,
             w2p_stack, w1p_stack, b2p_stack, b1_stack, cnt_in):
  nblk = O_NODES // NB
  full = lambda shape: pl.BlockSpec(shape, lambda i: tuple(0 for _ in shape))
  return pl.pallas_call(
      _prep_body,
      grid=(nblk,),
      in_specs=[
          pl.BlockSpec((NB, 1), lambda i: (i, 0)),
          pl.BlockSpec((NB, 1), lambda i: (i, 0)),
          full((ATTR_VOCAB, EMB)),
          full((ATTR_VOCAB, EMB)),
          pl.BlockSpec((NB, EMB), lambda i: (i, 0)),
          full((640, H)),
          full((1, H)),
          full((NLAYERS - 1, H, GDIM)),
          full((NLAYERS - 1, GDIM, H)),
          full((NLAYERS - 1, GDIM)),
          full((NLAYERS - 1, H)),
          pl.BlockSpec((NB, 2 * NW), lambda i: (i, 0)),
      ],
      out_specs=[
          pl.BlockSpec((NB, H), lambda i: (i, 0)),
          pl.BlockSpec((NB, H), lambda i: (i, 0)),
          pl.BlockSpec((NB, H), lambda i: (i, 0)),
          full((NLAYERS - 1, H, GDIM)),
          full((NLAYERS - 1, H)),
          pl.BlockSpec((NB, 1), lambda i: (i, 0)),
          pl.BlockSpec((NB, 1), lambda i: (i, 0)),
      ],
      out_shape=[
          jax.ShapeDtypeStruct((O_NODES, H), jnp.float32),
          jax.ShapeDtypeStruct((O_NODES, H), jnp.float32),
          jax.ShapeDtypeStruct((O_NODES, H), jnp.float32),
          jax.ShapeDtypeStruct((NLAYERS - 1, H, GDIM), jnp.float32),
          jax.ShapeDtypeStruct((NLAYERS - 1, H), jnp.float32),
          jax.ShapeDtypeStruct((O_NODES, 1), jnp.float32),
          jax.ShapeDtypeStruct((O_NODES, 1), jnp.float32),
      ],
  )(objs0, objs1, tab0, tab1, pred_emb, w1_0, b1_0,
    w2p_stack, w1p_stack, b2p_stack, b1_stack, cnt_in)


EBLK = 2048


def _edge_body_first(gsum_ref, mask_ref, m_ref, c_ref, hm_ref, gpn_ref):
  h = jnp.maximum(gsum_ref[...], 0.0)
  hm_ref[...] = h * mask_ref[...]
  gpn_ref[...] = _MM(h, m_ref[...]) + c_ref[...]


def _edge_body_mid(gsum_ref, gp_ref, mask_ref, m_ref, c_ref, hm_ref, gpn_ref):
  h = jnp.maximum(gsum_ref[...] + gp_ref[...], 0.0)
  hm_ref[...] = h * mask_ref[...]
  gpn_ref[...] = _MM(h, m_ref[...]) + c_ref[...]


def _edge_body_last(gsum_ref, gp_ref, mask_ref, hm_ref):
  h = jnp.maximum(gsum_ref[...] + gp_ref[...], 0.0)
  hm_ref[...] = h * mask_ref[...]


def _tc_edge(gsum, gp, maskc, m, c, last):
  nblk = TP // EBLK
  full = lambda shape: pl.BlockSpec(shape, lambda i: tuple(0 for _ in shape))
  row = pl.BlockSpec((EBLK, H), lambda i: (i, 0))
  mrow = pl.BlockSpec((EBLK, 1), lambda i: (i, 0))
  hm_shape = jax.ShapeDtypeStruct((TP, H), jnp.float32)
  if last:
    return pl.pallas_call(
        _edge_body_last, grid=(nblk,),
        in_specs=[row, row, mrow],
        out_specs=row, out_shape=hm_shape,
    )(gsum, gp, maskc)
  if gp is None:
    return pl.pallas_call(
        _edge_body_first, grid=(nblk,),
        in_specs=[row, mrow, full((H, GDIM)), full((1, GDIM))],
        out_specs=[row, row], out_shape=[hm_shape, hm_shape],
    )(gsum, maskc, m, c)
  return pl.pallas_call(
      _edge_body_mid, grid=(nblk,),
      in_specs=[row, row, mrow, full((H, GDIM)), full((1, GDIM))],
      out_specs=[row, row], out_shape=[hm_shape, hm_shape],
  )(gsum, gp, maskc, m, c)


def _node_body_mid(hs_ref, ho_ref, cs_ref, co_ref, w2s_ref, w2o_ref,
                   b2s_ref, b2o_ref, wn1_ref, bn1_ref, wn2_ref, bn2_ref,
                   w1s_ref, w1o_ref, as_ref, ao_ref):
  cs = cs_ref[...]
  co = co_ref[...]
  pooled = (_MM(hs_ref[...], w2s_ref[...]) + _MM(ho_ref[...], w2o_ref[...])
            + cs * b2s_ref[...] + co * b2o_ref[...])
  pooled = pooled / jnp.maximum(cs + co, 1.0)
  ov = _MM(jnp.maximum(_MM(pooled, wn1_ref[...]) + bn1_ref[...], 0.0),
           wn2_ref[...]) + bn2_ref[...]
  as_ref[...] = _MM(ov, w1s_ref[...])
  ao_ref[...] = _MM(ov, w1o_ref[...])


def _node_body_last(hs_ref, ho_ref, cs_ref, co_ref, w2s_ref, w2o_ref,
                    b2s_ref, b2o_ref, wn1_ref, bn1_ref, wn2_ref, bn2_ref,
                    wb1_ref, bb1_ref, wb2_ref, bb2_ref, ov_ref, box_ref):
  cs = cs_ref[...]
  co = co_ref[...]
  pooled = (_MM(hs_ref[...], w2s_ref[...]) + _MM(ho_ref[...], w2o_ref[...])
            + cs * b2s_ref[...] + co * b2o_ref[...])
  pooled = pooled / jnp.maximum(cs + co, 1.0)
  ov = _MM(jnp.maximum(_MM(pooled, wn1_ref[...]) + bn1_ref[...], 0.0),
           wn2_ref[...]) + bn2_ref[...]
  ov_ref[...] = ov
  box_ref[...] = _MM(jnp.maximum(_MM(ov, wb1_ref[...]) + bb1_ref[...], 0.0),
                     wb2_ref[...]) + bb2_ref[...]


def _tc_node(hs, ho, cnts, cnto, w2s, w2o, b2s, b2o, wn1, bn1, wn2, bn2,
             tail_ws, last):
  nblk = O_NODES // NB
  full = lambda shape: pl.BlockSpec(shape, lambda i: tuple(0 for _ in shape))
  row = pl.BlockSpec((NB, H), lambda i: (i, 0))
  col = pl.BlockSpec((NB, 1), lambda i: (i, 0))
  wspec = [full((H, H)), full((H, H)), full((1, H)), full((1, H)),
           full((H, H)), full((1, H)), full((H, GDIM)), full((1, GDIM))]
  if last:
    wb1, bb1, wb2, bb2 = tail_ws
    return pl.pallas_call(
        _node_body_last, grid=(nblk,),
        in_specs=[row, row, col, col] + wspec
        + [full((GDIM, H)), full((1, H)), full((H, 4)), full((1, 4))],
        out_specs=[row, pl.BlockSpec((NB, 4), lambda i: (i, 0))],
        out_shape=[jax.ShapeDtypeStruct((O_NODES, GDIM), jnp.float32),
                   jax.ShapeDtypeStruct((O_NODES, 4), jnp.float32)],
    )(hs, ho, cnts, cnto, w2s, w2o, b2s, b2o, wn1, bn1, wn2, bn2,
      wb1, bb1, wb2, bb2)
  w1s, w1o = tail_ws
  return pl.pallas_call(
      _node_body_mid, grid=(nblk,),
      in_specs=[row, row, col, col] + wspec
      + [full((GDIM, H)), full((GDIM, H))],
      out_specs=[row, row],
      out_shape=[jax.ShapeDtypeStruct((O_NODES, H), jnp.float32),
                 jax.ShapeDtypeStruct((O_NODES, H), jnp.float32)],
  )(hs, ho, cnts, cnto, w2s, w2o, b2s, b2o, wn1, bn1, wn2, bn2, w1s, w1o)


def kernel(objs, triplets, triplet_type, params):
  del triplet_type
  objs = objs.astype(jnp.int32)
  trip = triplets.astype(jnp.int32)
  pad = TP - T_EDGES
  s = jnp.pad(trip[:, 0], (0, pad))
  p = jnp.pad(trip[:, 1], (0, pad))
  o = jnp.pad(trip[:, 2], (0, pad))
  maskc = (p != 0).astype(jnp.float32)[:, None]
  idx2 = jnp.stack([s, o])

  layers = params['layers']
  w1_0 = layers[0]['W1']
  w2p_stack = jnp.stack([layers[l]['W2'][:, H:H + GDIM]
                         for l in range(NLAYERS - 1)])
  w1p_stack = jnp.stack([layers[l]['W1'][GDIM:2 * GDIM]
                         for l in range(1, NLAYERS)])
  b2p_stack = jnp.stack([layers[l]['b2'][H:H + GDIM]
                         for l in range(NLAYERS - 1)])
  b1_stack = jnp.stack([layers[l]['b1'] for l in range(1, NLAYERS)])

  cnt_parts = _sc_counts(s, o, p)
  a_s, a_o, p0, ms, cvecs, cnts, cnto = _tc_prep(
      objs[:, 0:1], objs[:, 1:2], params['attr_tab0'], params['attr_tab1'],
      params['pred_emb'], w1_0, params['layers'][0]['b1'][None, :],
      w2p_stack, w1p_stack, b2p_stack, b1_stack,
      cnt_parts.reshape(2 * NW, O_NODES).T)

  gp = None
  for l in range(NLAYERS):
    lp = layers[l]
    if l == 0:
      gsum = _sc_gather3(a_s, a_o, p0, s, o, p)
    else:
      gsum = _sc_gather2(a_s, a_o, s, o)
    last = (l == NLAYERS - 1)
    if last:
      hm = _tc_edge(gsum, gp, maskc, None, None, True)
    else:
      hm, gp = _tc_edge(gsum, gp, maskc, ms[l], cvecs[l:l + 1], False)
    tabs = _sc_scatter(hm, idx2)
    w2 = lp['W2']
    b2 = lp['b2']
    if last:
      tail = (params['Wb1'], params['bb1'][None, :],
              params['Wb2'], params['bb2'][None, :])
    else:
      w1n = layers[l + 1]['W1']
      tail = (w1n[0:GDIM], w1n[2 * GDIM:3 * GDIM])
    res = _tc_node(tabs[0], tabs[1], cnts, cnto,
                   w2[:, :H], w2[:, H + GDIM:],
                   b2[None, :H], b2[None, H + GDIM:],
                   lp['Wn1'], lp['bn1'][None, :],
                   lp['Wn2'], lp['bn2'][None, :], tail, last)
    if last:
      obj_out, boxes = res
    else:
      a_s, a_o = res

  return (obj_out, boxes)


# speedup vs baseline: 3.1295x; 1.4061x over previous
import functools

import jax
import jax.numpy as jnp
from jax import lax
from jax.experimental import pallas as pl
from jax.experimental.pallas import tpu as pltpu
from jax.experimental.pallas import tpu_sc as plsc

O_NODES = 10000
T_EDGES = 320000
EMB = 128
H = 128
GDIM = 128
NLAYERS = 5
ATTR_VOCAB = 200

NC = 2
NS = 16
NW = NC * NS
TP = 327680
CH = 128
EW_G = TP // NW
NCH_G = EW_G // CH
EW_S = TP // NS
NCH_S = EW_S // CH

_MM = functools.partial(jnp.dot, precision=lax.Precision.HIGHEST,
                        preferred_element_type=jnp.float32)

@functools.cache
def _mesh():
  return plsc.VectorSubcoreMesh(core_axis_name="c", subcore_axis_name="s",
                                num_cores=NC, num_subcores=NS)


def _wid():
  return lax.axis_index("s") * NC + lax.axis_index("c")


def _make_gather_body(ntab, ch):
  nch = EW_G // ch

  def body(*refs):
    tabs = refs[:ntab]
    idxs = refs[ntab:2 * ntab]
    out = refs[2 * ntab]
    scr = refs[2 * ntab + 1:]
    ixs = scr[:ntab]
    bufs = [scr[ntab + 2 * k: ntab + 2 * k + 2] for k in range(ntab)]
    oc = scr[3 * ntab: 3 * ntab + 2]
    semg = scr[3 * ntab + 2: 3 * ntab + 4]
    semw = scr[3 * ntab + 4: 3 * ntab + 6]
    base = _wid() * EW_G
    for k in range(ntab):
      pltpu.sync_copy(idxs[k].at[pl.ds(base, EW_G)], ixs[k])

    def g_copy(j, slot):
      return [pltpu.make_async_copy(
          tabs[k].at[ixs[k].at[pl.ds(j * ch, ch)]], bufs[k][slot], semg[slot])
              for k in range(ntab)]

    def g_issue(j, slot):
      for cp in g_copy(j, slot):
        cp.start()

    def g_wait(slot):
      for cp in g_copy(0, slot):
        cp.wait()

    def w_copy(j, slot):
      return pltpu.make_async_copy(oc[slot], out.at[pl.ds(base + j * ch, ch)],
                                   semw[slot])

    def do_adds(slot):
      dst = oc[slot]

      def addrow(r, c2):
        for q in range(8):
          sl = pl.ds(q * 16, 16)
          acc = bufs[0][slot][r, sl]
          for k in range(1, ntab):
            acc = acc + bufs[k][slot][r, sl]
          dst[r, sl] = acc
        return c2

      lax.fori_loop(0, ch, addrow, 0)

    g_issue(0, 0)
    g_issue(1, 1)

    def pair(i, carry):
      j0 = 2 * i
      for slot in range(2):
        g_wait(slot)

        @pl.when(i > 0)
        def _():
          w_copy(0, slot).wait()

        do_adds(slot)

        @pl.when(i < nch // 2 - 1)
        def _():
          g_issue(j0 + 2 + slot, slot)

        w_copy(j0 + slot, slot).start()
      return carry

    lax.fori_loop(0, nch // 2, pair, 0)
    w_copy(0, 0).wait()
    w_copy(0, 1).wait()

  return body


def _sc_gather(tabs, idxs):
  ntab = len(tabs)
  ch = CH if ntab == 2 else 64
  scratch = ([pltpu.VMEM((EW_G,), jnp.int32)] * ntab
             + [pltpu.VMEM((ch, H), jnp.float32)] * (2 * ntab)
             + [pltpu.VMEM((ch, H), jnp.float32)] * 2
             + [pltpu.SemaphoreType.DMA] * 4)
  k = pl.kernel(
      _make_gather_body(ntab, ch),
      out_type=jax.ShapeDtypeStruct((TP, H), jnp.float32),
      mesh=_mesh(),
      scratch_types=scratch,
  )
  return k(*tabs, *idxs)


def _sc_gather2(ta, tb, ia, ib):
  return _sc_gather((ta, tb), (ia, ib))


def _sc_gather3(ta, tb, tc_, ia, ib, ic):
  return _sc_gather((ta, tb, tc_), (ia, ib, ic))


def _scatter_body(hm, idx2, out, iv0, iv1, rv0, rv1, zb, acc,
                  semr0, semr1, sema0, sema1):
  c = lax.axis_index("c")
  sid = lax.axis_index("s")
  iv = (iv0, iv1)
  rv = (rv0, rv1)
  semr = (semr0, semr1)
  sema = (sema0, sema1)

  def zrow(r, c2):
    for q in range(8):
      zb[r, pl.ds(q * 16, 16)] = jnp.zeros((16,), jnp.float32)
    return c2

  lax.fori_loop(0, 64, zrow, 0)
  row0 = jnp.minimum(sid * 640, O_NODES - 640)

  def zcopy(kk, c2):
    pltpu.sync_copy(zb, acc.at[pl.ds(row0 + kk * 64, 64)])
    return c2

  lax.fori_loop(0, 10, zcopy, 0)
  plsc.subcore_barrier()

  base = sid * EW_S

  def r_copy(j, slot):
    off = base + j * CH
    return [pltpu.make_async_copy(idx2.at[c].at[pl.ds(off, CH)], iv[slot],
                                  semr[slot]),
            pltpu.make_async_copy(hm.at[pl.ds(off, CH)], rv[slot],
                                  semr[slot])]

  def r_issue(j, slot):
    for cp in r_copy(j, slot):
      cp.start()

  def r_wait(slot):
    for cp in r_copy(0, slot):
      cp.wait()

  def a_start(slot):
    pltpu.async_copy(rv[slot], acc.at[iv[slot]], sema[slot], add=True)

  def a_wait(slot):
    pltpu.make_async_copy(rv[slot], acc.at[iv[slot]], sema[slot]).wait()

  r_issue(0, 0)
  r_issue(1, 1)

  def pair(i, carry):
    j0 = 2 * i
    for slot in range(2):
      r_wait(slot)
      a_start(slot)
    for slot in range(2):

      @pl.when(i < NCH_S // 2 - 1)
      def _():
        a_wait(slot)
        r_issue(j0 + 2 + slot, slot)
    return carry

  lax.fori_loop(0, NCH_S // 2, pair, 0)
  a_wait(0)
  a_wait(1)
  plsc.subcore_barrier()
  pltpu.sync_copy(acc.at[pl.ds(row0, 640)], out.at[c].at[pl.ds(row0, 640)])


def _sc_scatter(hm, idx2):
  k = pl.kernel(
      _scatter_body,
      out_type=jax.ShapeDtypeStruct((2, O_NODES, H), jnp.float32),
      mesh=_mesh(),
      scratch_types=[
          pltpu.VMEM((CH,), jnp.int32),
          pltpu.VMEM((CH,), jnp.int32),
          pltpu.VMEM((CH, H), jnp.float32),
          pltpu.VMEM((CH, H), jnp.float32),
          pltpu.VMEM((64, H), jnp.float32),
          pltpu.VMEM_SHARED((O_NODES, H), jnp.float32),
          pltpu.SemaphoreType.DMA,
          pltpu.SemaphoreType.DMA,
          pltpu.SemaphoreType.DMA,
          pltpu.SemaphoreType.DMA,
      ],
  )
  return k(hm, idx2)


def _counts_body(sarr, oarr, parr, out, sb, ob, pb, cs, co):
  w = _wid()
  base = w * EW_G

  def zero(i, c2):
    sl = pl.ds(i * 16, 16)
    cs[sl] = jnp.zeros((16,), jnp.float32)
    co[sl] = jnp.zeros((16,), jnp.float32)
    return c2

  lax.fori_loop(0, O_NODES // 16, zero, 0)
  pltpu.sync_copy(sarr.at[pl.ds(base, EW_G)], sb)
  pltpu.sync_copy(oarr.at[pl.ds(base, EW_G)], ob)
  pltpu.sync_copy(parr.at[pl.ds(base, EW_G)], pb)

  def step(i, c2):
    sl = pl.ds(i * 16, 16)
    sv = sb[sl]
    ov = ob[sl]
    pv = pb[sl]
    m = jnp.where(pv != 0, 1.0, 0.0).astype(jnp.float32)
    plsc.addupdate_scatter(cs, [sv], m)
    plsc.addupdate_scatter(co, [ov], m)
    return c2

  lax.fori_loop(0, EW_G // 16, step, 0)
  pltpu.sync_copy(cs, out.at[0].at[w])
  pltpu.sync_copy(co, out.at[1].at[w])


def _sc_counts(sarr, oarr, parr):
  k = pl.kernel(
      _counts_body,
      out_type=jax.ShapeDtypeStruct((2, NW, O_NODES), jnp.float32),
      mesh=_mesh(),
      compiler_params=pltpu.CompilerParams(needs_layout_passes=False),
      scratch_types=[
          pltpu.VMEM((EW_G,), jnp.int32),
          pltpu.VMEM((EW_G,), jnp.int32),
          pltpu.VMEM((EW_G,), jnp.int32),
          pltpu.VMEM((O_NODES,), jnp.float32),
          pltpu.VMEM((O_NODES,), jnp.float32),
      ],
  )
  return k(sarr, oarr, parr)


NB = 2000


def _prep_body(o0_ref, o1_ref, tab0_ref, tab1_ref, pe_ref, w1_ref, b1_ref,
               w2ps_ref, w1ps_ref, b2ps_ref, b1s_ref, cin_ref,
               as_ref, ao_ref, p0_ref, ms_ref, cs_ref, cnts_ref, cnto_ref):
  w1 = w1_ref[...]
  tab0 = tab0_ref[...]
  tab1 = tab1_ref[...]
  t0s = _MM(tab0, w1[0:EMB])
  t1s = _MM(tab1, w1[EMB:2 * EMB])
  t0o = _MM(tab0, w1[384:384 + EMB])
  t1o = _MM(tab1, w1[384 + EMB:384 + 2 * EMB])
  iota = lax.broadcasted_iota(jnp.int32, (NB, ATTR_VOCAB), 1)
  oh0 = (o0_ref[...] == iota).astype(jnp.float32)
  oh1 = (o1_ref[...] == iota).astype(jnp.float32)
  as_ref[...] = _MM(oh0, t0s) + _MM(oh1, t1s)
  ao_ref[...] = _MM(oh0, t0o) + _MM(oh1, t1o)
  p0_ref[...] = _MM(pe_ref[...], w1[256:384]) + b1_ref[...]
  for l in range(NLAYERS - 1):
    ms_ref[l] = _MM(w2ps_ref[l], w1ps_ref[l])
    cs_ref[l:l + 1] = _MM(b2ps_ref[l:l + 1], w1ps_ref[l]) + b1s_ref[l:l + 1]
  cin = cin_ref[...]
  cnts_ref[...] = jnp.sum(cin[:, 0:NW], axis=1, keepdims=True)
  cnto_ref[...] = jnp.sum(cin[:, NW:2 * NW], axis=1, keepdims=True)


def _tc_prep(objs0, objs1, tab0, tab1, pred_emb, w1_0, b1_0,
             w2p_stack, w1p_stack, b2p_stack, b1_stack, cnt_in):
  nblk = O_NODES // NB
  full = lambda shape: pl.BlockSpec(shape, lambda i: tuple(0 for _ in shape))
  return pl.pallas_call(
      _prep_body,
      grid=(nblk,),
      in_specs=[
          pl.BlockSpec((NB, 1), lambda i: (i, 0)),
          pl.BlockSpec((NB, 1), lambda i: (i, 0)),
          full((ATTR_VOCAB, EMB)),
          full((ATTR_VOCAB, EMB)),
          pl.BlockSpec((NB, EMB), lambda i: (i, 0)),
          full((640, H)),
          full((1, H)),
          full((NLAYERS - 1, H, GDIM)),
          full((NLAYERS - 1, GDIM, H)),
          full((NLAYERS - 1, GDIM)),
          full((NLAYERS - 1, H)),
          pl.BlockSpec((NB, 2 * NW), lambda i: (i, 0)),
      ],
      out_specs=[
          pl.BlockSpec((NB, H), lambda i: (i, 0)),
          pl.BlockSpec((NB, H), lambda i: (i, 0)),
          pl.BlockSpec((NB, H), lambda i: (i, 0)),
          full((NLAYERS - 1, H, GDIM)),
          full((NLAYERS - 1, H)),
          pl.BlockSpec((NB, 1), lambda i: (i, 0)),
          pl.BlockSpec((NB, 1), lambda i: (i, 0)),
      ],
      out_shape=[
          jax.ShapeDtypeStruct((O_NODES, H), jnp.float32),
          jax.ShapeDtypeStruct((O_NODES, H), jnp.float32),
          jax.ShapeDtypeStruct((O_NODES, H), jnp.float32),
          jax.ShapeDtypeStruct((NLAYERS - 1, H, GDIM), jnp.float32),
          jax.ShapeDtypeStruct((NLAYERS - 1, H), jnp.float32),
          jax.ShapeDtypeStruct((O_NODES, 1), jnp.float32),
          jax.ShapeDtypeStruct((O_NODES, 1), jnp.float32),
      ],
  )(objs0, objs1, tab0, tab1, pred_emb, w1_0, b1_0,
    w2p_stack, w1p_stack, b2p_stack, b1_stack, cnt_in)


EBLK = 2048


def _edge_body_first(gsum_ref, mask_ref, m_ref, c_ref, hm_ref, gpn_ref):
  h = jnp.maximum(gsum_ref[...], 0.0)
  hm_ref[...] = h * mask_ref[...]
  gpn_ref[...] = _MM(h, m_ref[...]) + c_ref[...]


def _edge_body_mid(gsum_ref, gp_ref, mask_ref, m_ref, c_ref, hm_ref, gpn_ref):
  h = jnp.maximum(gsum_ref[...] + gp_ref[...], 0.0)
  hm_ref[...] = h * mask_ref[...]
  gpn_ref[...] = _MM(h, m_ref[...]) + c_ref[...]


def _edge_body_last(gsum_ref, gp_ref, mask_ref, hm_ref):
  h = jnp.maximum(gsum_ref[...] + gp_ref[...], 0.0)
  hm_ref[...] = h * mask_ref[...]


def _tc_edge(gsum, gp, maskc, m, c, last):
  nblk = TP // EBLK
  full = lambda shape: pl.BlockSpec(shape, lambda i: tuple(0 for _ in shape))
  row = pl.BlockSpec((EBLK, H), lambda i: (i, 0))
  mrow = pl.BlockSpec((EBLK, 1), lambda i: (i, 0))
  hm_shape = jax.ShapeDtypeStruct((TP, H), jnp.float32)
  if last:
    return pl.pallas_call(
        _edge_body_last, grid=(nblk,),
        in_specs=[row, row, mrow],
        out_specs=row, out_shape=hm_shape,
    )(gsum, gp, maskc)
  if gp is None:
    return pl.pallas_call(
        _edge_body_first, grid=(nblk,),
        in_specs=[row, mrow, full((H, GDIM)), full((1, GDIM))],
        out_specs=[row, row], out_shape=[hm_shape, hm_shape],
    )(gsum, maskc, m, c)
  return pl.pallas_call(
      _edge_body_mid, grid=(nblk,),
      in_specs=[row, row, mrow, full((H, GDIM)), full((1, GDIM))],
      out_specs=[row, row], out_shape=[hm_shape, hm_shape],
  )(gsum, gp, maskc, m, c)


def _node_body_mid(hs_ref, ho_ref, cs_ref, co_ref, w2s_ref, w2o_ref,
                   b2s_ref, b2o_ref, wn1_ref, bn1_ref, wn2_ref, bn2_ref,
                   w1s_ref, w1o_ref, as_ref, ao_ref):
  cs = cs_ref[...]
  co = co_ref[...]
  pooled = (_MM(hs_ref[...], w2s_ref[...]) + _MM(ho_ref[...], w2o_ref[...])
            + cs * b2s_ref[...] + co * b2o_ref[...])
  pooled = pooled / jnp.maximum(cs + co, 1.0)
  ov = _MM(jnp.maximum(_MM(pooled, wn1_ref[...]) + bn1_ref[...], 0.0),
           wn2_ref[...]) + bn2_ref[...]
  as_ref[...] = _MM(ov, w1s_ref[...])
  ao_ref[...] = _MM(ov, w1o_ref[...])


def _node_body_last(hs_ref, ho_ref, cs_ref, co_ref, w2s_ref, w2o_ref,
                    b2s_ref, b2o_ref, wn1_ref, bn1_ref, wn2_ref, bn2_ref,
                    wb1_ref, bb1_ref, wb2_ref, bb2_ref, ov_ref, box_ref):
  cs = cs_ref[...]
  co = co_ref[...]
  pooled = (_MM(hs_ref[...], w2s_ref[...]) + _MM(ho_ref[...], w2o_ref[...])
            + cs * b2s_ref[...] + co * b2o_ref[...])
  pooled = pooled / jnp.maximum(cs + co, 1.0)
  ov = _MM(jnp.maximum(_MM(pooled, wn1_ref[...]) + bn1_ref[...], 0.0),
           wn2_ref[...]) + bn2_ref[...]
  ov_ref[...] = ov
  box_ref[...] = _MM(jnp.maximum(_MM(ov, wb1_ref[...]) + bb1_ref[...], 0.0),
                     wb2_ref[...]) + bb2_ref[...]


def _tc_node(hs, ho, cnts, cnto, w2s, w2o, b2s, b2o, wn1, bn1, wn2, bn2,
             tail_ws, last):
  nblk = O_NODES // NB
  full = lambda shape: pl.BlockSpec(shape, lambda i: tuple(0 for _ in shape))
  row = pl.BlockSpec((NB, H), lambda i: (i, 0))
  col = pl.BlockSpec((NB, 1), lambda i: (i, 0))
  wspec = [full((H, H)), full((H, H)), full((1, H)), full((1, H)),
           full((H, H)), full((1, H)), full((H, GDIM)), full((1, GDIM))]
  if last:
    wb1, bb1, wb2, bb2 = tail_ws
    return pl.pallas_call(
        _node_body_last, grid=(nblk,),
        in_specs=[row, row, col, col] + wspec
        + [full((GDIM, H)), full((1, H)), full((H, 4)), full((1, 4))],
        out_specs=[row, pl.BlockSpec((NB, 4), lambda i: (i, 0))],
        out_shape=[jax.ShapeDtypeStruct((O_NODES, GDIM), jnp.float32),
                   jax.ShapeDtypeStruct((O_NODES, 4), jnp.float32)],
    )(hs, ho, cnts, cnto, w2s, w2o, b2s, b2o, wn1, bn1, wn2, bn2,
      wb1, bb1, wb2, bb2)
  w1s, w1o = tail_ws
  return pl.pallas_call(
      _node_body_mid, grid=(nblk,),
      in_specs=[row, row, col, col] + wspec
      + [full((GDIM, H)), full((GDIM, H))],
      out_specs=[row, row],
      out_shape=[jax.ShapeDtypeStruct((O_NODES, H), jnp.float32),
                 jax.ShapeDtypeStruct((O_NODES, H), jnp.float32)],
  )(hs, ho, cnts, cnto, w2s, w2o, b2s, b2o, wn1, bn1, wn2, bn2, w1s, w1o)


def kernel(objs, triplets, triplet_type, params):
  del triplet_type
  objs = objs.astype(jnp.int32)
  trip = triplets.astype(jnp.int32)
  pad = TP - T_EDGES
  s = jnp.pad(trip[:, 0], (0, pad))
  p = jnp.pad(trip[:, 1], (0, pad))
  o = jnp.pad(trip[:, 2], (0, pad))
  maskc = (p != 0).astype(jnp.float32)[:, None]
  idx2 = jnp.stack([s, o])

  layers = params['layers']
  w1_0 = layers[0]['W1']
  w2p_stack = jnp.stack([layers[l]['W2'][:, H:H + GDIM]
                         for l in range(NLAYERS - 1)])
  w1p_stack = jnp.stack([layers[l]['W1'][GDIM:2 * GDIM]
                         for l in range(1, NLAYERS)])
  b2p_stack = jnp.stack([layers[l]['b2'][H:H + GDIM]
                         for l in range(NLAYERS - 1)])
  b1_stack = jnp.stack([layers[l]['b1'] for l in range(1, NLAYERS)])

  cnt_parts = _sc_counts(s, o, p)
  a_s, a_o, p0, ms, cvecs, cnts, cnto = _tc_prep(
      objs[:, 0:1], objs[:, 1:2], params['attr_tab0'], params['attr_tab1'],
      params['pred_emb'], w1_0, params['layers'][0]['b1'][None, :],
      w2p_stack, w1p_stack, b2p_stack, b1_stack,
      cnt_parts.reshape(2 * NW, O_NODES).T)

  gp = None
  for l in range(NLAYERS):
    lp = layers[l]
    if l == 0:
      gsum = _sc_gather3(a_s, a_o, p0, s, o, p)
    else:
      gsum = _sc_gather2(a_s, a_o, s, o)
    last = (l == NLAYERS - 1)
    if last:
      hm = _tc_edge(gsum, gp, maskc, None, None, True)
    else:
      hm, gp = _tc_edge(gsum, gp, maskc, ms[l], cvecs[l:l + 1], False)
    tabs = _sc_scatter(hm, idx2)
    w2 = lp['W2']
    b2 = lp['b2']
    if last:
      tail = (params['Wb1'], params['bb1'][None, :],
              params['Wb2'], params['bb2'][None, :])
    else:
      w1n = layers[l + 1]['W1']
      tail = (w1n[0:GDIM], w1n[2 * GDIM:3 * GDIM])
    res = _tc_node(tabs[0], tabs[1], cnts, cnto,
                   w2[:, :H], w2[:, H + GDIM:],
                   b2[None, :H], b2[None, H + GDIM:],
                   lp['Wn1'], lp['bn1'][None, :],
                   lp['Wn2'], lp['bn2'][None, :], tail, last)
    if last:
      obj_out, boxes = res
    else:
      a_s, a_o = res

  return (obj_out, boxes)


# speedup vs baseline: 3.3730x; 1.0778x over previous
import functools

import jax
import jax.numpy as jnp
from jax import lax
from jax.experimental import pallas as pl
from jax.experimental.pallas import tpu as pltpu
from jax.experimental.pallas import tpu_sc as plsc

O_NODES = 10000
T_EDGES = 320000
EMB = 128
H = 128
GDIM = 128
NLAYERS = 5
ATTR_VOCAB = 200

NC = 2
NS = 16
NW = NC * NS
TP = 327680
CH = 128
EW_G = TP // NW
NCH_G = EW_G // CH
EW_S = TP // NS
SCH = 64
NCH_S = EW_S // SCH

_MM = functools.partial(jnp.dot, precision=lax.Precision.HIGHEST,
                        preferred_element_type=jnp.float32)

@functools.cache
def _mesh():
  return plsc.VectorSubcoreMesh(core_axis_name="c", subcore_axis_name="s",
                                num_cores=NC, num_subcores=NS)


def _wid():
  return lax.axis_index("s") * NC + lax.axis_index("c")


NSLOT = 4


def _make_gather_body(ntab, ch, nslot):
  nch = EW_G // ch
  NSLOT = nslot

  def body(*refs):
    tabs = refs[:ntab]
    idxs = refs[ntab:2 * ntab]
    out = refs[2 * ntab]
    scr = refs[2 * ntab + 1:]
    ixs = scr[:ntab]
    bufs = [scr[ntab + NSLOT * k: ntab + NSLOT * (k + 1)]
            for k in range(ntab)]
    semg = scr[ntab * (NSLOT + 1): ntab * (NSLOT + 1) + NSLOT]
    semw = scr[ntab * (NSLOT + 1) + NSLOT: ntab * (NSLOT + 1) + 2 * NSLOT]
    base = _wid() * EW_G
    for k in range(ntab):
      pltpu.sync_copy(idxs[k].at[pl.ds(base, EW_G)], ixs[k])

    def g_copy(j, slot):
      return [pltpu.make_async_copy(
          tabs[k].at[ixs[k].at[pl.ds(j * ch, ch)]], bufs[k][slot], semg[slot])
              for k in range(ntab)]

    def g_issue(j, slot):
      for cp in g_copy(j, slot):
        cp.start()

    def g_wait(slot):
      for cp in g_copy(0, slot):
        cp.wait()

    def w_copy(j, slot):
      return pltpu.make_async_copy(bufs[-1][slot],
                                   out.at[pl.ds(base + j * ch, ch)],
                                   semw[slot])

    def do_adds(slot):
      dst = bufs[-1][slot]

      def addrow(r, c2):
        for q in range(8):
          sl = pl.ds(q * 16, 16)
          acc = bufs[0][slot][r, sl]
          for k in range(1, ntab - 1):
            acc = acc + bufs[k][slot][r, sl]
          dst[r, sl] = dst[r, sl] + acc
        return c2

      lax.fori_loop(0, ch, addrow, 0)

    for slot in range(NSLOT - 1):
      g_issue(slot, slot)

    def quad(i, carry):
      for u in range(NSLOT):
        j = NSLOT * i + u
        g_wait(u)
        do_adds(u)
        w_copy(j, u).start()
        t = (u + NSLOT - 1) % NSLOT

        @pl.when(j > 0)
        def _():
          w_copy(0, t).wait()

        @pl.when(j + NSLOT - 1 < nch)
        def _():
          g_issue(j + NSLOT - 1, t)
      return carry

    lax.fori_loop(0, nch // NSLOT, quad, 0)
    w_copy(0, (nch - 1) % NSLOT).wait()

  return body


def _sc_gather(tabs, idxs):
  ntab = len(tabs)
  ch = 64 if ntab == 2 else 32
  scratch = ([pltpu.VMEM((EW_G,), jnp.int32)] * ntab
             + [pltpu.VMEM((ch, H), jnp.float32)] * (NSLOT * ntab)
             + [pltpu.SemaphoreType.DMA] * (2 * NSLOT))
  k = pl.kernel(
      _make_gather_body(ntab, ch, NSLOT),
      out_type=jax.ShapeDtypeStruct((TP, H), jnp.float32),
      mesh=_mesh(),
      scratch_types=scratch,
  )
  return k(*tabs, *idxs)


def _sc_gather2(ta, tb, ia, ib):
  return _sc_gather((ta, tb), (ia, ib))


def _sc_gather3(ta, tb, tc_, ia, ib, ic):
  return _sc_gather((ta, tb, tc_), (ia, ib, ic))


def _scatter_body(hm, idx2, out, *scr):
  c = lax.axis_index("c")
  sid = lax.axis_index("s")
  iv = scr[0:NSLOT]
  rv = scr[NSLOT:2 * NSLOT]
  zb = scr[2 * NSLOT]
  acc = scr[2 * NSLOT + 1]
  semr = scr[2 * NSLOT + 2: 2 * NSLOT + 2 + NSLOT]
  sema = scr[2 * NSLOT + 2 + NSLOT: 2 * NSLOT + 2 + 2 * NSLOT]

  def zrow(r, c2):
    for q in range(8):
      zb[r, pl.ds(q * 16, 16)] = jnp.zeros((16,), jnp.float32)
    return c2

  lax.fori_loop(0, 64, zrow, 0)
  row0 = jnp.minimum(sid * 640, O_NODES - 640)

  def zcopy(kk, c2):
    pltpu.sync_copy(zb, acc.at[pl.ds(row0 + kk * 64, 64)])
    return c2

  lax.fori_loop(0, 10, zcopy, 0)
  plsc.subcore_barrier()

  base = sid * EW_S

  def r_copy(j, slot):
    off = base + j * SCH
    return [pltpu.make_async_copy(idx2.at[c].at[pl.ds(off, SCH)], iv[slot],
                                  semr[slot]),
            pltpu.make_async_copy(hm.at[pl.ds(off, SCH)], rv[slot],
                                  semr[slot])]

  def r_issue(j, slot):
    for cp in r_copy(j, slot):
      cp.start()

  def r_wait(slot):
    for cp in r_copy(0, slot):
      cp.wait()

  def a_start(slot):
    pltpu.async_copy(rv[slot], acc.at[iv[slot]], sema[slot], add=True)

  def a_wait(slot):
    pltpu.make_async_copy(rv[slot], acc.at[iv[slot]], sema[slot]).wait()

  for slot in range(NSLOT - 1):
    r_issue(slot, slot)

  def quad(i, carry):
    for u in range(NSLOT):
      j = NSLOT * i + u
      r_wait(u)
      a_start(u)
      t = (u + NSLOT - 1) % NSLOT

      @pl.when(j > 0)
      def _():
        a_wait(t)

      @pl.when(j + NSLOT - 1 < NCH_S)
      def _():
        r_issue(j + NSLOT - 1, t)
    return carry

  lax.fori_loop(0, NCH_S // NSLOT, quad, 0)
  a_wait((NCH_S - 1) % NSLOT)
  plsc.subcore_barrier()
  pltpu.sync_copy(acc.at[pl.ds(row0, 640)], out.at[c].at[pl.ds(row0, 640)])


def _sc_scatter(hm, idx2):
  k = pl.kernel(
      _scatter_body,
      out_type=jax.ShapeDtypeStruct((2, O_NODES, H), jnp.float32),
      mesh=_mesh(),
      scratch_types=(
          [pltpu.VMEM((SCH,), jnp.int32)] * NSLOT
          + [pltpu.VMEM((SCH, H), jnp.float32)] * NSLOT
          + [pltpu.VMEM((64, H), jnp.float32),
             pltpu.VMEM_SHARED((O_NODES, H), jnp.float32)]
          + [pltpu.SemaphoreType.DMA] * (2 * NSLOT)
      ),
  )
  return k(hm, idx2)


def _counts_body(sarr, oarr, parr, out, sb, ob, pb, cs, co):
  w = _wid()
  base = w * EW_G

  def zero(i, c2):
    sl = pl.ds(i * 16, 16)
    cs[sl] = jnp.zeros((16,), jnp.float32)
    co[sl] = jnp.zeros((16,), jnp.float32)
    return c2

  lax.fori_loop(0, O_NODES // 16, zero, 0)
  pltpu.sync_copy(sarr.at[pl.ds(base, EW_G)], sb)
  pltpu.sync_copy(oarr.at[pl.ds(base, EW_G)], ob)
  pltpu.sync_copy(parr.at[pl.ds(base, EW_G)], pb)

  def step(i, c2):
    sl = pl.ds(i * 16, 16)
    sv = sb[sl]
    ov = ob[sl]
    pv = pb[sl]
    m = jnp.where(pv != 0, 1.0, 0.0).astype(jnp.float32)
    plsc.addupdate_scatter(cs, [sv], m)
    plsc.addupdate_scatter(co, [ov], m)
    return c2

  lax.fori_loop(0, EW_G // 16, step, 0)
  pltpu.sync_copy(cs, out.at[0].at[w])
  pltpu.sync_copy(co, out.at[1].at[w])


def _sc_counts(sarr, oarr, parr):
  k = pl.kernel(
      _counts_body,
      out_type=jax.ShapeDtypeStruct((2, NW, O_NODES), jnp.float32),
      mesh=_mesh(),
      compiler_params=pltpu.CompilerParams(needs_layout_passes=False),
      scratch_types=[
          pltpu.VMEM((EW_G,), jnp.int32),
          pltpu.VMEM((EW_G,), jnp.int32),
          pltpu.VMEM((EW_G,), jnp.int32),
          pltpu.VMEM((O_NODES,), jnp.float32),
          pltpu.VMEM((O_NODES,), jnp.float32),
      ],
  )
  return k(sarr, oarr, parr)


NB = 2000


def _prep_body(o0_ref, o1_ref, tab0_ref, tab1_ref, pe_ref, w1_ref, b1_ref,
               w2ps_ref, w1ps_ref, b2ps_ref, b1s_ref, cin_ref,
               as_ref, ao_ref, p0_ref, ms_ref, cs_ref, cnts_ref, cnto_ref):
  w1 = w1_ref[...]
  tab0 = tab0_ref[...]
  tab1 = tab1_ref[...]
  t0s = _MM(tab0, w1[0:EMB])
  t1s = _MM(tab1, w1[EMB:2 * EMB])
  t0o = _MM(tab0, w1[384:384 + EMB])
  t1o = _MM(tab1, w1[384 + EMB:384 + 2 * EMB])
  iota = lax.broadcasted_iota(jnp.int32, (NB, ATTR_VOCAB), 1)
  oh0 = (o0_ref[...] == iota).astype(jnp.float32)
  oh1 = (o1_ref[...] == iota).astype(jnp.float32)
  as_ref[...] = _MM(oh0, t0s) + _MM(oh1, t1s)
  ao_ref[...] = _MM(oh0, t0o) + _MM(oh1, t1o)
  p0_ref[...] = _MM(pe_ref[...], w1[256:384]) + b1_ref[...]
  for l in range(NLAYERS - 1):
    ms_ref[l] = _MM(w2ps_ref[l], w1ps_ref[l])
    cs_ref[l:l + 1] = _MM(b2ps_ref[l:l + 1], w1ps_ref[l]) + b1s_ref[l:l + 1]
  cin = cin_ref[...]
  cnts_ref[...] = jnp.sum(cin[:, 0:NW], axis=1, keepdims=True)
  cnto_ref[...] = jnp.sum(cin[:, NW:2 * NW], axis=1, keepdims=True)


def _tc_prep(objs0, objs1, tab0, tab1, pred_emb, w1_0, b1_0,
             w2p_stack, w1p_stack, b2p_stack, b1_stack, cnt_in):
  nblk = O_NODES // NB
  full = lambda shape: pl.BlockSpec(shape, lambda i: tuple(0 for _ in shape))
  return pl.pallas_call(
      _prep_body,
      grid=(nblk,),
      in_specs=[
          pl.BlockSpec((NB, 1), lambda i: (i, 0)),
          pl.BlockSpec((NB, 1), lambda i: (i, 0)),
          full((ATTR_VOCAB, EMB)),
          full((ATTR_VOCAB, EMB)),
          pl.BlockSpec((NB, EMB), lambda i: (i, 0)),
          full((640, H)),
          full((1, H)),
          full((NLAYERS - 1, H, GDIM)),
          full((NLAYERS - 1, GDIM, H)),
          full((NLAYERS - 1, GDIM)),
          full((NLAYERS - 1, H)),
          pl.BlockSpec((NB, 2 * NW), lambda i: (i, 0)),
      ],
      out_specs=[
          pl.BlockSpec((NB, H), lambda i: (i, 0)),
          pl.BlockSpec((NB, H), lambda i: (i, 0)),
          pl.BlockSpec((NB, H), lambda i: (i, 0)),
          full((NLAYERS - 1, H, GDIM)),
          full((NLAYERS - 1, H)),
          pl.BlockSpec((NB, 1), lambda i: (i, 0)),
          pl.BlockSpec((NB, 1), lambda i: (i, 0)),
      ],
      out_shape=[
          jax.ShapeDtypeStruct((O_NODES, H), jnp.float32),
          jax.ShapeDtypeStruct((O_NODES, H), jnp.float32),
          jax.ShapeDtypeStruct((O_NODES, H), jnp.float32),
          jax.ShapeDtypeStruct((NLAYERS - 1, H, GDIM), jnp.float32),
          jax.ShapeDtypeStruct((NLAYERS - 1, H), jnp.float32),
          jax.ShapeDtypeStruct((O_NODES, 1), jnp.float32),
          jax.ShapeDtypeStruct((O_NODES, 1), jnp.float32),
      ],
  )(objs0, objs1, tab0, tab1, pred_emb, w1_0, b1_0,
    w2p_stack, w1p_stack, b2p_stack, b1_stack, cnt_in)


EBLK = 2048


def _edge_body_first(gsum_ref, mask_ref, m_ref, c_ref, hm_ref, gpn_ref):
  h = jnp.maximum(gsum_ref[...], 0.0)
  hm_ref[...] = h * mask_ref[...]
  gpn_ref[...] = _MM(h, m_ref[...]) + c_ref[...]


def _edge_body_mid(gsum_ref, gp_ref, mask_ref, m_ref, c_ref, hm_ref, gpn_ref):
  h = jnp.maximum(gsum_ref[...] + gp_ref[...], 0.0)
  hm_ref[...] = h * mask_ref[...]
  gpn_ref[...] = _MM(h, m_ref[...]) + c_ref[...]


def _edge_body_last(gsum_ref, gp_ref, mask_ref, hm_ref):
  h = jnp.maximum(gsum_ref[...] + gp_ref[...], 0.0)
  hm_ref[...] = h * mask_ref[...]


def _tc_edge(gsum, gp, maskc, m, c, last):
  nblk = TP // EBLK
  full = lambda shape: pl.BlockSpec(shape, lambda i: tuple(0 for _ in shape))
  row = pl.BlockSpec((EBLK, H), lambda i: (i, 0))
  mrow = pl.BlockSpec((EBLK, 1), lambda i: (i, 0))
  hm_shape = jax.ShapeDtypeStruct((TP, H), jnp.float32)
  if last:
    return pl.pallas_call(
        _edge_body_last, grid=(nblk,),
        in_specs=[row, row, mrow],
        out_specs=row, out_shape=hm_shape,
    )(gsum, gp, maskc)
  if gp is None:
    return pl.pallas_call(
        _edge_body_first, grid=(nblk,),
        in_specs=[row, mrow, full((H, GDIM)), full((1, GDIM))],
        out_specs=[row, row], out_shape=[hm_shape, hm_shape],
    )(gsum, maskc, m, c)
  return pl.pallas_call(
      _edge_body_mid, grid=(nblk,),
      in_specs=[row, row, mrow, full((H, GDIM)), full((1, GDIM))],
      out_specs=[row, row], out_shape=[hm_shape, hm_shape],
  )(gsum, gp, maskc, m, c)


def _node_body_mid(hs_ref, ho_ref, cs_ref, co_ref, w2s_ref, w2o_ref,
                   b2s_ref, b2o_ref, wn1_ref, bn1_ref, wn2_ref, bn2_ref,
                   w1s_ref, w1o_ref, as_ref, ao_ref):
  cs = cs_ref[...]
  co = co_ref[...]
  pooled = (_MM(hs_ref[...], w2s_ref[...]) + _MM(ho_ref[...], w2o_ref[...])
            + cs * b2s_ref[...] + co * b2o_ref[...])
  pooled = pooled / jnp.maximum(cs + co, 1.0)
  ov = _MM(jnp.maximum(_MM(pooled, wn1_ref[...]) + bn1_ref[...], 0.0),
           wn2_ref[...]) + bn2_ref[...]
  as_ref[...] = _MM(ov, w1s_ref[...])
  ao_ref[...] = _MM(ov, w1o_ref[...])


def _node_body_last(hs_ref, ho_ref, cs_ref, co_ref, w2s_ref, w2o_ref,
                    b2s_ref, b2o_ref, wn1_ref, bn1_ref, wn2_ref, bn2_ref,
                    wb1_ref, bb1_ref, wb2_ref, bb2_ref, ov_ref, box_ref):
  cs = cs_ref[...]
  co = co_ref[...]
  pooled = (_MM(hs_ref[...], w2s_ref[...]) + _MM(ho_ref[...], w2o_ref[...])
            + cs * b2s_ref[...] + co * b2o_ref[...])
  pooled = pooled / jnp.maximum(cs + co, 1.0)
  ov = _MM(jnp.maximum(_MM(pooled, wn1_ref[...]) + bn1_ref[...], 0.0),
           wn2_ref[...]) + bn2_ref[...]
  ov_ref[...] = ov
  box_ref[...] = _MM(jnp.maximum(_MM(ov, wb1_ref[...]) + bb1_ref[...], 0.0),
                     wb2_ref[...]) + bb2_ref[...]


def _tc_node(hs, ho, cnts, cnto, w2s, w2o, b2s, b2o, wn1, bn1, wn2, bn2,
             tail_ws, last):
  nblk = O_NODES // NB
  full = lambda shape: pl.BlockSpec(shape, lambda i: tuple(0 for _ in shape))
  row = pl.BlockSpec((NB, H), lambda i: (i, 0))
  col = pl.BlockSpec((NB, 1), lambda i: (i, 0))
  wspec = [full((H, H)), full((H, H)), full((1, H)), full((1, H)),
           full((H, H)), full((1, H)), full((H, GDIM)), full((1, GDIM))]
  if last:
    wb1, bb1, wb2, bb2 = tail_ws
    return pl.pallas_call(
        _node_body_last, grid=(nblk,),
        in_specs=[row, row, col, col] + wspec
        + [full((GDIM, H)), full((1, H)), full((H, 4)), full((1, 4))],
        out_specs=[row, pl.BlockSpec((NB, 4), lambda i: (i, 0))],
        out_shape=[jax.ShapeDtypeStruct((O_NODES, GDIM), jnp.float32),
                   jax.ShapeDtypeStruct((O_NODES, 4), jnp.float32)],
    )(hs, ho, cnts, cnto, w2s, w2o, b2s, b2o, wn1, bn1, wn2, bn2,
      wb1, bb1, wb2, bb2)
  w1s, w1o = tail_ws
  return pl.pallas_call(
      _node_body_mid, grid=(nblk,),
      in_specs=[row, row, col, col] + wspec
      + [full((GDIM, H)), full((GDIM, H))],
      out_specs=[row, row],
      out_shape=[jax.ShapeDtypeStruct((O_NODES, H), jnp.float32),
                 jax.ShapeDtypeStruct((O_NODES, H), jnp.float32)],
  )(hs, ho, cnts, cnto, w2s, w2o, b2s, b2o, wn1, bn1, wn2, bn2, w1s, w1o)


def kernel(objs, triplets, triplet_type, params):
  del triplet_type
  objs = objs.astype(jnp.int32)
  trip = triplets.astype(jnp.int32)
  pad = TP - T_EDGES
  s = jnp.pad(trip[:, 0], (0, pad))
  p = jnp.pad(trip[:, 1], (0, pad))
  o = jnp.pad(trip[:, 2], (0, pad))
  maskc = (p != 0).astype(jnp.float32)[:, None]
  idx2 = jnp.stack([s, o])

  layers = params['layers']
  w1_0 = layers[0]['W1']
  w2p_stack = jnp.stack([layers[l]['W2'][:, H:H + GDIM]
                         for l in range(NLAYERS - 1)])
  w1p_stack = jnp.stack([layers[l]['W1'][GDIM:2 * GDIM]
                         for l in range(1, NLAYERS)])
  b2p_stack = jnp.stack([layers[l]['b2'][H:H + GDIM]
                         for l in range(NLAYERS - 1)])
  b1_stack = jnp.stack([layers[l]['b1'] for l in range(1, NLAYERS)])

  cnt_parts = _sc_counts(s, o, p)
  a_s, a_o, p0, ms, cvecs, cnts, cnto = _tc_prep(
      objs[:, 0:1], objs[:, 1:2], params['attr_tab0'], params['attr_tab1'],
      params['pred_emb'], w1_0, params['layers'][0]['b1'][None, :],
      w2p_stack, w1p_stack, b2p_stack, b1_stack,
      cnt_parts.reshape(2 * NW, O_NODES).T)

  gp = None
  for l in range(NLAYERS):
    lp = layers[l]
    if l == 0:
      gsum = _sc_gather3(a_s, a_o, p0, s, o, p)
    else:
      gsum = _sc_gather2(a_s, a_o, s, o)
    last = (l == NLAYERS - 1)
    if last:
      hm = _tc_edge(gsum, gp, maskc, None, None, True)
    else:
      hm, gp = _tc_edge(gsum, gp, maskc, ms[l], cvecs[l:l + 1], False)
    tabs = _sc_scatter(hm, idx2)
    w2 = lp['W2']
    b2 = lp['b2']
    if last:
      tail = (params['Wb1'], params['bb1'][None, :],
              params['Wb2'], params['bb2'][None, :])
    else:
      w1n = layers[l + 1]['W1']
      tail = (w1n[0:GDIM], w1n[2 * GDIM:3 * GDIM])
    res = _tc_node(tabs[0], tabs[1], cnts, cnto,
                   w2[:, :H], w2[:, H + GDIM:],
                   b2[None, :H], b2[None, H + GDIM:],
                   lp['Wn1'], lp['bn1'][None, :],
                   lp['Wn2'], lp['bn2'][None, :], tail, last)
    if last:
      obj_out, boxes = res
    else:
      a_s, a_o = res

  return (obj_out, boxes)
